# baseline (speedup 1.0000x reference)
import functools

import jax
import jax.numpy as jnp
from jax import lax
from jax.experimental import pallas as pl
from jax.experimental.pallas import tpu as pltpu
from jax.experimental.pallas import tpu_sc as plsc

N = 10000
E = 320000
D = 128
H = 128
G = 64

NC = 2
NS = 16
NW = NC * NS

CHUNK = 80
HT = 63
T = 2 * HT
E_PAD = NW * T * CHUNK
NA = N + 8
NBUF = 2

SHIFT = 14


def _unpack_chunk(pk_ref, j, sbuf, dbuf, b):
    for k in range(CHUNK // 16):
        v = pk_ref[j, pl.ds(k * 16, 16)]
        sbuf[b, pl.ds(k * 16, 16)] = lax.shift_right_logical(v, SHIFT)
        dbuf[b, pl.ds(k * 16, 16)] = lax.bitwise_and(v, (1 << SHIFT) - 1)


def _segsum_body(h_hbm, pk_hbm, out_hbm, pk, sbuf, dbuf, rows, agg_ref, sems):
    c = lax.axis_index("c")
    s = lax.axis_index("s")
    wid = s * NC + c

    def _zero_row(j, _):
        for k in range(H // 16):
            rows[0, j, pl.ds(k * 16, 16)] = jnp.zeros((16,), jnp.float32)
        return _

    lax.fori_loop(0, CHUNK, _zero_row, None)
    base = s * 624

    zbuf = rows.at[0]
    for i in range(7):
        pltpu.sync_copy(zbuf, agg_ref.at[pl.ds(base + i * 80, 80)])

    @pl.when(s < NS - 1)
    def _():
        pltpu.sync_copy(zbuf.at[pl.ds(0, 64)],
                        agg_ref.at[pl.ds(base + 560, 64)])

    @pl.when(s == NS - 1)
    def _():
        pltpu.sync_copy(zbuf, agg_ref.at[pl.ds(base + 560, 80)])

    plsc.subcore_barrier()

    pltpu.sync_copy(pk_hbm.at[wid], pk)
    _unpack_chunk(pk, 0, sbuf, dbuf, 0)
    pltpu.async_copy(h_hbm.at[sbuf.at[0]], rows.at[0], sems.at[0])

    @pl.loop(0, T - 2, step=2)
    def _(g0):
        for b in range(2):
            l = g0 + b
            bn = (b + 1) % 2
            _unpack_chunk(pk, l + 1, sbuf, dbuf, bn)
            pltpu.async_copy(h_hbm.at[sbuf.at[bn]], rows.at[bn], sems.at[bn])
            pltpu.make_async_copy(h_hbm.at[sbuf.at[b]], rows.at[b],
                                  sems.at[b]).wait()
            pltpu.sync_copy(rows.at[b], agg_ref.at[dbuf.at[b]], add=True)

    _unpack_chunk(pk, T - 1, sbuf, dbuf, 1)
    pltpu.async_copy(h_hbm.at[sbuf.at[1]], rows.at[1], sems.at[1])
    pltpu.make_async_copy(h_hbm.at[sbuf.at[0]], rows.at[0], sems.at[0]).wait()
    pltpu.sync_copy(rows.at[0], agg_ref.at[dbuf.at[0]], add=True)
    pltpu.make_async_copy(h_hbm.at[sbuf.at[1]], rows.at[1], sems.at[1]).wait()
    pltpu.sync_copy(rows.at[1], agg_ref.at[dbuf.at[1]], add=True)
    plsc.subcore_barrier()

    for i in range(7):
        pltpu.sync_copy(agg_ref.at[pl.ds(base + i * 80, 80)],
                        out_hbm.at[c].at[pl.ds(base + i * 80, 80)])

    @pl.when(s < NS - 1)
    def _():
        pltpu.sync_copy(agg_ref.at[pl.ds(base + 560, 64)],
                        out_hbm.at[c].at[pl.ds(base + 560, 64)])

    @pl.when(s == NS - 1)
    def _():
        pltpu.sync_copy(agg_ref.at[pl.ds(base + 560, 80)],
                        out_hbm.at[c].at[pl.ds(base + 560, 80)])


@functools.partial(
    pl.kernel,
    out_type=jax.ShapeDtypeStruct((NC, N, H), jnp.float32),
    mesh=plsc.VectorSubcoreMesh(core_axis_name="c", subcore_axis_name="s"),
    scratch_types=[
        pltpu.VMEM((T, CHUNK), jnp.int32),
        pltpu.VMEM((NBUF, CHUNK), jnp.int32),
        pltpu.VMEM((NBUF, CHUNK), jnp.int32),
        pltpu.VMEM((NBUF, CHUNK, H), jnp.float32),
        pltpu.VMEM_SHARED((NA, H), jnp.float32),
        pltpu.SemaphoreType.DMA((NBUF,)),
    ],
)
def _segsum(h_hbm, pk_hbm, out_hbm, pk, sbuf, dbuf, rows, agg, sems):
    _segsum_body(h_hbm, pk_hbm, out_hbm, pk, sbuf, dbuf, rows, agg, sems)


BN = 1000


def _conv_body(h_ref, p_ref, wr_ref, wn_ref, b_ref, o_ref):
    p = p_ref[0] + p_ref[1]
    acc = jnp.dot(h_ref[...], wr_ref[...], preferred_element_type=jnp.float32)
    acc += jnp.dot(p, wn_ref[...], preferred_element_type=jnp.float32)
    o_ref[...] = jnp.maximum(acc + b_ref[...], 0.0)


_conv = pl.pallas_call(
    _conv_body,
    grid=(N // BN,),
    in_specs=[
        pl.BlockSpec((BN, H), lambda i: (i, 0)),
        pl.BlockSpec((NC, BN, H), lambda i: (0, i, 0)),
        pl.BlockSpec((H, H), lambda i: (0, 0)),
        pl.BlockSpec((H, H), lambda i: (0, 0)),
        pl.BlockSpec((1, H), lambda i: (0, 0)),
    ],
    out_specs=pl.BlockSpec((BN, H), lambda i: (i, 0)),
    out_shape=jax.ShapeDtypeStruct((N, H), jnp.float32),
)


def _head_body(h_ref, b3_ref, wf1_ref, bf1_ref, wf2_ref, bf2_ref, o_ref,
               pooled):
    i = pl.program_id(0)

    @pl.when(i == 0)
    def _():
        pooled[...] = jnp.zeros((G, H), jnp.float32)

    seg = lax.broadcasted_iota(jnp.int32, (G, BN), 0)
    mask = (b3_ref[0, 0] == seg).astype(jnp.float32)
    pooled[...] += jnp.dot(mask, h_ref[...], preferred_element_type=jnp.float32)

    @pl.when(i == pl.num_programs(0) - 1)
    def _():
        h2 = jnp.maximum(
            jnp.dot(pooled[...], wf1_ref[...],
                    preferred_element_type=jnp.float32) + bf1_ref[...], 0.0)
        o_ref[...] = jnp.dot(h2, wf2_ref[...],
                             preferred_element_type=jnp.float32) + bf2_ref[...]


_head = pl.pallas_call(
    _head_body,
    grid=(N // BN,),
    in_specs=[
        pl.BlockSpec((BN, H), lambda i: (i, 0)),
        pl.BlockSpec((1, 1, BN), lambda i: (i, 0, 0)),
        pl.BlockSpec((H, H), lambda i: (0, 0)),
        pl.BlockSpec((1, H), lambda i: (0, 0)),
        pl.BlockSpec((H, 1), lambda i: (0, 0)),
        pl.BlockSpec((1, 1), lambda i: (0, 0)),
    ],
    out_specs=pl.BlockSpec((G, 1), lambda i: (0, 0)),
    out_shape=jax.ShapeDtypeStruct((G, 1), jnp.float32),
    scratch_shapes=[pltpu.VMEM((G, H), jnp.float32)],
)


def kernel(x, edge_index, batch,
           W1r, W1n, b1, W2r, W2n, b2, W3r, W3n, b3,
           W4r, W4n, b4, W5r, W5n, b5, Wf1, bf1, Wf2, bf2):
    packed = edge_index[0] * (1 << SHIFT) + edge_index[1]
    pad = jnp.full((E_PAD - E,), N, dtype=jnp.int32)
    packed = jnp.concatenate([packed, pad]).reshape(NW, T, CHUNK)
    batch3d = batch.reshape(N // BN, 1, BN)

    h = x
    layers = [(W1r, W1n, b1), (W2r, W2n, b2), (W3r, W3n, b3),
              (W4r, W4n, b4), (W5r, W5n, b5)]
    for Wr, Wn, b in layers:
        parts = _segsum(h, packed)
        h = _conv(h, parts, Wr, Wn, b.reshape(1, H))
    return _head(h, batch3d, Wf1, bf1.reshape(1, H), Wf2,
                 bf2.reshape(1, 1))


# speedup vs baseline: 1.8852x; 1.8852x over previous
import functools

import jax
import jax.numpy as jnp
from jax import lax
from jax.experimental import pallas as pl
from jax.experimental.pallas import tpu as pltpu
from jax.experimental.pallas import tpu_sc as plsc

N = 10000
E = 320000
D = 128
H = 128
G = 64

NC = 2
NS = 16
NW = NC * NS

CHUNK = 80
T = 125
NA = N
NBUF = 2

SHIFT = 14


def _unpack_chunk(pk_ref, j, sbuf, dbuf, b):
    for k in range(CHUNK // 16):
        v = pk_ref[j, pl.ds(k * 16, 16)]
        sbuf[b, pl.ds(k * 16, 16)] = lax.shift_right_logical(v, SHIFT)
        dbuf[b, pl.ds(k * 16, 16)] = lax.bitwise_and(v, (1 << SHIFT) - 1)


def _segsum_body(h_hbm, pk_hbm, out_hbm, pk, sbuf, dbuf, rows, agg_ref, sems):
    c = lax.axis_index("c")
    s = lax.axis_index("s")
    wid = s * NC + c

    def _zero_row(j, _):
        for k in range(H // 16):
            rows[0, j, pl.ds(k * 16, 16)] = jnp.zeros((16,), jnp.float32)
        return _

    lax.fori_loop(0, CHUNK, _zero_row, None)
    base = s * 624

    zbuf = rows.at[0]
    for i in range(7):
        pltpu.sync_copy(zbuf, agg_ref.at[pl.ds(base + i * 80, 80)])

    @pl.when(s < NS - 1)
    def _():
        pltpu.sync_copy(zbuf.at[pl.ds(0, 64)],
                        agg_ref.at[pl.ds(base + 560, 64)])

    @pl.when(s == NS - 1)
    def _():
        pltpu.sync_copy(zbuf, agg_ref.at[pl.ds(base + 560, 80)])

    plsc.subcore_barrier()

    pltpu.sync_copy(pk_hbm.at[wid], pk)
    _unpack_chunk(pk, 0, sbuf, dbuf, 0)
    pltpu.async_copy(h_hbm.at[sbuf.at[0]], rows.at[0], sems.at[0])

    @pl.loop(0, T - 1, step=2)
    def _(g0):
        for b in range(2):
            l = g0 + b
            bn = (b + 1) % 2
            _unpack_chunk(pk, l + 1, sbuf, dbuf, bn)
            pltpu.async_copy(h_hbm.at[sbuf.at[bn]], rows.at[bn], sems.at[bn])
            pltpu.make_async_copy(h_hbm.at[sbuf.at[b]], rows.at[b],
                                  sems.at[b]).wait()
            pltpu.sync_copy(rows.at[b], agg_ref.at[dbuf.at[b]], add=True)

    pltpu.make_async_copy(h_hbm.at[sbuf.at[0]], rows.at[0], sems.at[0]).wait()
    pltpu.sync_copy(rows.at[0], agg_ref.at[dbuf.at[0]], add=True)
    plsc.subcore_barrier()

    for i in range(7):
        pltpu.sync_copy(agg_ref.at[pl.ds(base + i * 80, 80)],
                        out_hbm.at[c].at[pl.ds(base + i * 80, 80)])

    @pl.when(s < NS - 1)
    def _():
        pltpu.sync_copy(agg_ref.at[pl.ds(base + 560, 64)],
                        out_hbm.at[c].at[pl.ds(base + 560, 64)])

    @pl.when(s == NS - 1)
    def _():
        pltpu.sync_copy(agg_ref.at[pl.ds(base + 560, 80)],
                        out_hbm.at[c].at[pl.ds(base + 560, 80)])


@functools.partial(
    pl.kernel,
    out_type=jax.ShapeDtypeStruct((NC, N, H), jnp.float32),
    mesh=plsc.VectorSubcoreMesh(core_axis_name="c", subcore_axis_name="s"),
    scratch_types=[
        pltpu.VMEM((T, CHUNK), jnp.int32),
        pltpu.VMEM((NBUF, CHUNK), jnp.int32),
        pltpu.VMEM((NBUF, CHUNK), jnp.int32),
        pltpu.VMEM((NBUF, CHUNK, H), jnp.float32),
        pltpu.VMEM_SHARED((NA, H), jnp.float32),
        pltpu.SemaphoreType.DMA((NBUF,)),
    ],
)
def _segsum(h_hbm, pk_hbm, out_hbm, pk, sbuf, dbuf, rows, agg, sems):
    _segsum_body(h_hbm, pk_hbm, out_hbm, pk, sbuf, dbuf, rows, agg, sems)


BN = 1000


def _conv_body(h_ref, p_ref, wr_ref, wn_ref, b_ref, o_ref):
    p = p_ref[0] + p_ref[1]
    acc = jnp.dot(h_ref[...], wr_ref[...], preferred_element_type=jnp.float32)
    acc += jnp.dot(p, wn_ref[...], preferred_element_type=jnp.float32)
    o_ref[...] = jnp.maximum(acc + b_ref[...], 0.0)


_conv = pl.pallas_call(
    _conv_body,
    grid=(N // BN,),
    in_specs=[
        pl.BlockSpec((BN, H), lambda i: (i, 0)),
        pl.BlockSpec((NC, BN, H), lambda i: (0, i, 0)),
        pl.BlockSpec((H, H), lambda i: (0, 0)),
        pl.BlockSpec((H, H), lambda i: (0, 0)),
        pl.BlockSpec((1, H), lambda i: (0, 0)),
    ],
    out_specs=pl.BlockSpec((BN, H), lambda i: (i, 0)),
    out_shape=jax.ShapeDtypeStruct((N, H), jnp.float32),
)


def _head_body(h_ref, b3_ref, wf1_ref, bf1_ref, wf2_ref, bf2_ref, o_ref,
               pooled):
    i = pl.program_id(0)

    @pl.when(i == 0)
    def _():
        pooled[...] = jnp.zeros((G, H), jnp.float32)

    seg = lax.broadcasted_iota(jnp.int32, (G, BN), 0)
    mask = (b3_ref[0, 0] == seg).astype(jnp.float32)
    pooled[...] += jnp.dot(mask, h_ref[...], preferred_element_type=jnp.float32)

    @pl.when(i == pl.num_programs(0) - 1)
    def _():
        h2 = jnp.maximum(
            jnp.dot(pooled[...], wf1_ref[...],
                    preferred_element_type=jnp.float32) + bf1_ref[...], 0.0)
        o_ref[...] = jnp.dot(h2, wf2_ref[...],
                             preferred_element_type=jnp.float32) + bf2_ref[...]


_head = pl.pallas_call(
    _head_body,
    grid=(N // BN,),
    in_specs=[
        pl.BlockSpec((BN, H), lambda i: (i, 0)),
        pl.BlockSpec((1, 1, BN), lambda i: (i, 0, 0)),
        pl.BlockSpec((H, H), lambda i: (0, 0)),
        pl.BlockSpec((1, H), lambda i: (0, 0)),
        pl.BlockSpec((H, 1), lambda i: (0, 0)),
        pl.BlockSpec((1, 1), lambda i: (0, 0)),
    ],
    out_specs=pl.BlockSpec((G, 1), lambda i: (0, 0)),
    out_shape=jax.ShapeDtypeStruct((G, 1), jnp.float32),
    scratch_shapes=[pltpu.VMEM((G, H), jnp.float32)],
)


def kernel(x, edge_index, batch,
           W1r, W1n, b1, W2r, W2n, b2, W3r, W3n, b3,
           W4r, W4n, b4, W5r, W5n, b5, Wf1, bf1, Wf2, bf2):
    packed = (edge_index[0] * (1 << SHIFT) + edge_index[1]).reshape(
        NW, T, CHUNK)
    batch3d = batch.reshape(N // BN, 1, BN)

    h = x
    layers = [(W1r, W1n, b1), (W2r, W2n, b2), (W3r, W3n, b3),
              (W4r, W4n, b4), (W5r, W5n, b5)]
    for Wr, Wn, b in layers:
        parts = _segsum(h, packed)
        h = _conv(h, parts, Wr, Wn, b.reshape(1, H))
    return _head(h, batch3d, Wf1, bf1.reshape(1, H), Wf2,
                 bf2.reshape(1, 1))


# speedup vs baseline: 1.9031x; 1.0095x over previous
import functools

import jax
import jax.numpy as jnp
from jax import lax
from jax.experimental import pallas as pl
from jax.experimental.pallas import tpu as pltpu
from jax.experimental.pallas import tpu_sc as plsc

N = 10000
E = 320000
D = 128
H = 128
G = 64

NC = 2
NS = 16
NW = NC * NS

CHUNK = 80
T = 125
NA = N
NBUF = 2

SHIFT = 14


def _unpack_chunk(pk_ref, j, sbuf, dbuf, b):
    for k in range(CHUNK // 16):
        v = pk_ref[j, pl.ds(k * 16, 16)]
        sbuf[b, pl.ds(k * 16, 16)] = lax.shift_right_logical(v, SHIFT)
        dbuf[b, pl.ds(k * 16, 16)] = lax.bitwise_and(v, (1 << SHIFT) - 1)


def _segsum_body(h_hbm, pk_hbm, out_hbm, pk, sbuf, dbuf, rows, agg_ref, sems):
    c = lax.axis_index("c")
    s = lax.axis_index("s")
    wid = s * NC + c

    def _zero_row(j, _):
        for k in range(H // 16):
            rows[1, j, pl.ds(k * 16, 16)] = jnp.zeros((16,), jnp.float32)
        return _

    lax.fori_loop(0, CHUNK, _zero_row, None)
    base = s * 624

    pltpu.sync_copy(pk_hbm.at[wid], pk)
    _unpack_chunk(pk, 0, sbuf, dbuf, 0)
    pltpu.async_copy(h_hbm.at[sbuf.at[0]], rows.at[0], sems.at[0])

    zbuf = rows.at[1]
    for i in range(7):
        pltpu.sync_copy(zbuf, agg_ref.at[pl.ds(base + i * 80, 80)])

    @pl.when(s < NS - 1)
    def _():
        pltpu.sync_copy(zbuf.at[pl.ds(0, 64)],
                        agg_ref.at[pl.ds(base + 560, 64)])

    @pl.when(s == NS - 1)
    def _():
        pltpu.sync_copy(zbuf, agg_ref.at[pl.ds(base + 560, 80)])

    plsc.subcore_barrier()


    @pl.loop(0, T - 1, step=2)
    def _(g0):
        for b in range(2):
            l = g0 + b
            bn = (b + 1) % 2
            _unpack_chunk(pk, l + 1, sbuf, dbuf, bn)
            pltpu.async_copy(h_hbm.at[sbuf.at[bn]], rows.at[bn], sems.at[bn])
            pltpu.make_async_copy(h_hbm.at[sbuf.at[b]], rows.at[b],
                                  sems.at[b]).wait()
            pltpu.sync_copy(rows.at[b], agg_ref.at[dbuf.at[b]], add=True)

    pltpu.make_async_copy(h_hbm.at[sbuf.at[0]], rows.at[0], sems.at[0]).wait()
    pltpu.sync_copy(rows.at[0], agg_ref.at[dbuf.at[0]], add=True)
    plsc.subcore_barrier()

    for i in range(7):
        pltpu.sync_copy(agg_ref.at[pl.ds(base + i * 80, 80)],
                        out_hbm.at[c].at[pl.ds(base + i * 80, 80)])

    @pl.when(s < NS - 1)
    def _():
        pltpu.sync_copy(agg_ref.at[pl.ds(base + 560, 64)],
                        out_hbm.at[c].at[pl.ds(base + 560, 64)])

    @pl.when(s == NS - 1)
    def _():
        pltpu.sync_copy(agg_ref.at[pl.ds(base + 560, 80)],
                        out_hbm.at[c].at[pl.ds(base + 560, 80)])


@functools.partial(
    pl.kernel,
    out_type=jax.ShapeDtypeStruct((NC, N, H), jnp.float32),
    mesh=plsc.VectorSubcoreMesh(core_axis_name="c", subcore_axis_name="s"),
    scratch_types=[
        pltpu.VMEM((T, CHUNK), jnp.int32),
        pltpu.VMEM((NBUF, CHUNK), jnp.int32),
        pltpu.VMEM((NBUF, CHUNK), jnp.int32),
        pltpu.VMEM((NBUF, CHUNK, H), jnp.float32),
        pltpu.VMEM_SHARED((NA, H), jnp.float32),
        pltpu.SemaphoreType.DMA((NBUF,)),
    ],
)
def _segsum(h_hbm, pk_hbm, out_hbm, pk, sbuf, dbuf, rows, agg, sems):
    _segsum_body(h_hbm, pk_hbm, out_hbm, pk, sbuf, dbuf, rows, agg, sems)


BN = 1000


def _conv_body(h_ref, p_ref, wr_ref, wn_ref, b_ref, o_ref):
    p = p_ref[0] + p_ref[1]
    acc = jnp.dot(h_ref[...], wr_ref[...], preferred_element_type=jnp.float32)
    acc += jnp.dot(p, wn_ref[...], preferred_element_type=jnp.float32)
    o_ref[...] = jnp.maximum(acc + b_ref[...], 0.0)


_conv = pl.pallas_call(
    _conv_body,
    grid=(N // BN,),
    in_specs=[
        pl.BlockSpec((BN, H), lambda i: (i, 0)),
        pl.BlockSpec((NC, BN, H), lambda i: (0, i, 0)),
        pl.BlockSpec((H, H), lambda i: (0, 0)),
        pl.BlockSpec((H, H), lambda i: (0, 0)),
        pl.BlockSpec((1, H), lambda i: (0, 0)),
    ],
    out_specs=pl.BlockSpec((BN, H), lambda i: (i, 0)),
    out_shape=jax.ShapeDtypeStruct((N, H), jnp.float32),
)


def _head_body(h_ref, b3_ref, wf1_ref, bf1_ref, wf2_ref, bf2_ref, o_ref,
               pooled):
    i = pl.program_id(0)

    @pl.when(i == 0)
    def _():
        pooled[...] = jnp.zeros((G, H), jnp.float32)

    seg = lax.broadcasted_iota(jnp.int32, (G, BN), 0)
    mask = (b3_ref[0, 0] == seg).astype(jnp.float32)
    pooled[...] += jnp.dot(mask, h_ref[...], preferred_element_type=jnp.float32)

    @pl.when(i == pl.num_programs(0) - 1)
    def _():
        h2 = jnp.maximum(
            jnp.dot(pooled[...], wf1_ref[...],
                    preferred_element_type=jnp.float32) + bf1_ref[...], 0.0)
        o_ref[...] = jnp.dot(h2, wf2_ref[...],
                             preferred_element_type=jnp.float32) + bf2_ref[...]


_head = pl.pallas_call(
    _head_body,
    grid=(N // BN,),
    in_specs=[
        pl.BlockSpec((BN, H), lambda i: (i, 0)),
        pl.BlockSpec((1, 1, BN), lambda i: (i, 0, 0)),
        pl.BlockSpec((H, H), lambda i: (0, 0)),
        pl.BlockSpec((1, H), lambda i: (0, 0)),
        pl.BlockSpec((H, 1), lambda i: (0, 0)),
        pl.BlockSpec((1, 1), lambda i: (0, 0)),
    ],
    out_specs=pl.BlockSpec((G, 1), lambda i: (0, 0)),
    out_shape=jax.ShapeDtypeStruct((G, 1), jnp.float32),
    scratch_shapes=[pltpu.VMEM((G, H), jnp.float32)],
)


def kernel(x, edge_index, batch,
           W1r, W1n, b1, W2r, W2n, b2, W3r, W3n, b3,
           W4r, W4n, b4, W5r, W5n, b5, Wf1, bf1, Wf2, bf2):
    packed = (edge_index[0] * (1 << SHIFT) + edge_index[1]).reshape(
        NW, T, CHUNK)
    batch3d = batch.reshape(N // BN, 1, BN)

    h = x
    layers = [(W1r, W1n, b1), (W2r, W2n, b2), (W3r, W3n, b3),
              (W4r, W4n, b4), (W5r, W5n, b5)]
    for Wr, Wn, b in layers:
        parts = _segsum(h, packed)
        h = _conv(h, parts, Wr, Wn, b.reshape(1, H))
    return _head(h, batch3d, Wf1, bf1.reshape(1, H), Wf2,
                 bf2.reshape(1, 1))
